# Initial kernel scaffold; baseline (speedup 1.0000x reference)
import jax
import jax.numpy as jnp
from jax.experimental import pallas as pl


def kernel(x1, edge_index1, b1, status, W1, bc1, W2, bc2, W3, bc3, Wlin, blin):
    raise NotImplementedError("write your pallas kernel here")


# speedup vs baseline: 33.6500x; 33.6500x over previous
import functools

import jax
import jax.numpy as jnp
from jax import lax
from jax.experimental import pallas as pl
from jax.experimental.pallas import tpu as pltpu
from jax.experimental.pallas import tpu_sc as plsc

N = 10000
E = 640000
NG = 64
NPAD = 10240
NCORES = 2
NSUB = 16
TILES = NCORES * NSUB
CHUNK = 128
IDXBLK = 16
NBLK = 10
NCHUNK = IDXBLK * NBLK
EPAD = TILES * NCHUNK * CHUNK
RPT = NPAD // NSUB
RB = 1024
GRID = NPAD // RB


def _sc_mesh():
    return plsc.VectorSubcoreMesh(
        core_axis_name="c", subcore_axis_name="s",
        num_cores=NCORES, num_subcores=NSUB)


def _make_edge_pass(D):
    @functools.partial(
        pl.kernel,
        out_type=jax.ShapeDtypeStruct((NCORES, NSUB, RPT, D), jnp.float32),
        mesh=_sc_mesh(),
        compiler_params=pltpu.CompilerParams(use_tc_tiling_on_sc=False),
        scratch_types=[
            pltpu.VMEM_SHARED((NPAD, D), jnp.float32),
            pltpu.VMEM((IDXBLK, CHUNK), jnp.int32),
            pltpu.VMEM((IDXBLK, CHUNK), jnp.int32),
            pltpu.VMEM((CHUNK, D), jnp.float32),
            pltpu.VMEM((CHUNK, D), jnp.float32),
            pltpu.SemaphoreType.DMA,
            pltpu.SemaphoreType.DMA,
        ],
    )
    def edge_pass(table, srcp, dstp, zeros, out,
                  accum, src_v, dst_v, buf0, buf1, gsem0, gsem1):
        c = lax.axis_index("c")
        s = lax.axis_index("s")
        wid = c * NSUB + s
        pltpu.sync_copy(zeros, accum.at[pl.ds(s * RPT, RPT)])
        plsc.subcore_barrier()

        def blk_body(blk, _):
            pltpu.sync_copy(srcp.at[wid, blk], src_v)
            pltpu.sync_copy(dstp.at[wid, blk], dst_v)
            pltpu.async_copy(table.at[src_v.at[0]], buf0, gsem0).wait()

            def body(i, _):
                j = i * 2
                pltpu.async_copy(table.at[src_v.at[j + 1]], buf1, gsem1)
                pltpu.sync_copy(buf0, accum.at[dst_v.at[j]], add=True)
                pltpu.make_async_copy(
                    table.at[src_v.at[j + 1]], buf1, gsem1).wait()

                @pl.when(j + 2 < IDXBLK)
                def _():
                    pltpu.async_copy(table.at[src_v.at[j + 2]], buf0, gsem0)

                pltpu.sync_copy(buf1, accum.at[dst_v.at[j + 1]], add=True)

                @pl.when(j + 2 < IDXBLK)
                def _():
                    pltpu.make_async_copy(
                        table.at[src_v.at[j + 2]], buf0, gsem0).wait()

                return 0

            lax.fori_loop(0, IDXBLK // 2, body, 0)
            return 0

        lax.fori_loop(0, NBLK, blk_body, 0)
        plsc.subcore_barrier()
        pltpu.sync_copy(accum.at[pl.ds(s * RPT, RPT)], out.at[c, s])

    return edge_pass


@functools.partial(
    pl.kernel,
    out_type=jax.ShapeDtypeStruct((NCORES, NSUB, RPT, 8), jnp.float32),
    mesh=_sc_mesh(),
    compiler_params=pltpu.CompilerParams(use_tc_tiling_on_sc=False),
    scratch_types=[
        pltpu.VMEM_SHARED((NPAD, 8), jnp.float32),
        pltpu.VMEM((NCHUNK, CHUNK), jnp.int32),
        pltpu.VMEM((CHUNK, 8), jnp.float32),
    ],
)
def _deg_pass(dstp, zeros, ones, out, accum, dst_v, ones_v):
    c = lax.axis_index("c")
    s = lax.axis_index("s")
    wid = c * NSUB + s
    pltpu.sync_copy(zeros, accum.at[pl.ds(s * RPT, RPT)])
    pltpu.sync_copy(dstp.at[wid], dst_v)
    pltpu.sync_copy(ones, ones_v)
    plsc.subcore_barrier()

    def body(j, _):
        pltpu.sync_copy(ones_v, accum.at[dst_v.at[j]], add=True)
        return 0

    lax.fori_loop(0, NCHUNK, body, 0)
    plsc.subcore_barrier()
    pltpu.sync_copy(accum.at[pl.ds(s * RPT, RPT)], out.at[c, s])


def _tc_a_body(x_ref, w1_ref, degp_ref, u1_ref, dinv_ref):
    deg = degp_ref[0] + degp_ref[1] + 1.0
    dinv = lax.rsqrt(deg)
    dinv_ref[...] = dinv
    xw = jnp.dot(x_ref[...], w1_ref[...], preferred_element_type=jnp.float32)
    u1_ref[...] = xw * dinv[:, :1]


def _tc_b_body(sp_ref, u1_ref, dinv_ref, bc1_ref, u2_ref):
    dinv = dinv_ref[:, :1]
    t = dinv * (sp_ref[0] + sp_ref[1] + u1_ref[...]) + bc1_ref[...]
    u2_ref[...] = dinv * jnp.maximum(t, 0.0)


def _tc_c_body(sp_ref, u2_ref, dinv_ref, w2_ref, bc2_ref, u3_ref):
    dinv = dinv_ref[:, :1]
    t = dinv * (sp_ref[0] + sp_ref[1] + u2_ref[...])
    h = jnp.maximum(
        jnp.dot(t, w2_ref[...], preferred_element_type=jnp.float32)
        + bc2_ref[...], 0.0)
    u3_ref[...] = dinv * h


def _tc_d_body(sp_ref, u3_ref, dinv_ref, b1_ref, w3_ref, bc3_ref,
               wlin_ref, blin_ref, out_ref, g_acc, cnt_acc):
    i = pl.program_id(0)

    @pl.when(i == 0)
    def _():
        g_acc[...] = jnp.zeros_like(g_acc)
        cnt_acc[...] = jnp.zeros_like(cnt_acc)

    z = dinv_ref[:, :1] * (sp_ref[0] + sp_ref[1] + u3_ref[...])
    gids = lax.broadcasted_iota(jnp.int32, (1, NG), 1)
    onehot = (b1_ref[...] == gids).astype(jnp.float32)
    g_acc[...] += lax.dot_general(
        onehot, z, (((0,), (0,)), ((), ())),
        preferred_element_type=jnp.float32)
    cnt_acc[...] += lax.dot_general(
        onehot, jnp.ones((RB, 8), jnp.float32), (((0,), (0,)), ((), ())),
        preferred_element_type=jnp.float32)

    @pl.when(i == GRID - 1)
    def _():
        cnt = cnt_acc[:, :1]
        psum = jnp.dot(g_acc[...], w3_ref[...],
                       preferred_element_type=jnp.float32) + cnt * bc3_ref[...]
        pooled = psum / jnp.maximum(cnt, 1.0)
        out_ref[...] = jnp.dot(pooled, wlin_ref[...],
                               preferred_element_type=jnp.float32) + blin_ref[...]


def _row_spec(d):
    return pl.BlockSpec((RB, d), lambda i: (i, 0))


def _full_spec(shape):
    nd = len(shape)
    return pl.BlockSpec(shape, lambda i, _nd=nd: (0,) * _nd)


def _partial_spec(d):
    return pl.BlockSpec((NCORES, RB, d), lambda i: (0, i, 0))


def kernel(x1, edge_index1, b1, status, W1, bc1, W2, bc2, W3, bc3, Wlin, blin):
    f32 = jnp.float32
    src = edge_index1[0]
    dst = edge_index1[1]
    npad_extra = NPAD - N
    pad_e = EPAD - E
    padidx = (N + jnp.arange(pad_e, dtype=jnp.int32) % npad_extra)
    srcp = jnp.concatenate([src, padidx]).reshape(TILES, NBLK, IDXBLK, CHUNK)
    dstp = jnp.concatenate([dst, padidx]).reshape(TILES, NBLK, IDXBLK, CHUNK)
    dstp_flat = dstp.reshape(TILES, NCHUNK, CHUNK)
    xpad = jnp.concatenate([x1, jnp.zeros((npad_extra, x1.shape[1]), f32)])
    b1p = jnp.concatenate([b1, jnp.full((npad_extra,), NG, jnp.int32)])
    b1p = b1p.reshape(NPAD, 1)
    zeros8 = jnp.zeros((RPT, 8), f32)
    zeros64 = jnp.zeros((RPT, 64), f32)
    zeros128 = jnp.zeros((RPT, 128), f32)
    ones8 = jnp.ones((CHUNK, 8), f32)

    degp = _deg_pass(dstp_flat, zeros8, ones8).reshape(NCORES, NPAD, 8)

    tc_a = pl.pallas_call(
        _tc_a_body,
        grid=(GRID,),
        in_specs=[_row_spec(128), _full_spec((128, 64)), _partial_spec(8)],
        out_specs=[_row_spec(64), _row_spec(8)],
        out_shape=[jax.ShapeDtypeStruct((NPAD, 64), f32),
                   jax.ShapeDtypeStruct((NPAD, 8), f32)],
    )
    U1, dinv8 = tc_a(xpad, W1, degp)

    edge64 = _make_edge_pass(64)
    edge128 = _make_edge_pass(128)

    S1 = edge64(U1, srcp, dstp, zeros64).reshape(NCORES, NPAD, 64)

    tc_b = pl.pallas_call(
        _tc_b_body,
        grid=(GRID,),
        in_specs=[_partial_spec(64), _row_spec(64), _row_spec(8),
                  _full_spec((1, 64))],
        out_specs=_row_spec(64),
        out_shape=jax.ShapeDtypeStruct((NPAD, 64), f32),
    )
    U2 = tc_b(S1, U1, dinv8, bc1.reshape(1, 64))

    S2 = edge64(U2, srcp, dstp, zeros64).reshape(NCORES, NPAD, 64)

    tc_c = pl.pallas_call(
        _tc_c_body,
        grid=(GRID,),
        in_specs=[_partial_spec(64), _row_spec(64), _row_spec(8),
                  _full_spec((64, 128)), _full_spec((1, 128))],
        out_specs=_row_spec(128),
        out_shape=jax.ShapeDtypeStruct((NPAD, 128), f32),
    )
    U3 = tc_c(S2, U2, dinv8, W2, bc2.reshape(1, 128))

    S3 = edge128(U3, srcp, dstp, zeros128).reshape(NCORES, NPAD, 128)

    tc_d = pl.pallas_call(
        _tc_d_body,
        grid=(GRID,),
        in_specs=[_partial_spec(128), _row_spec(128), _row_spec(8),
                  pl.BlockSpec((RB, 1), lambda i: (i, 0)),
                  _full_spec((128, 256)), _full_spec((1, 256)),
                  _full_spec((256, 1)), _full_spec((1, 1))],
        out_specs=pl.BlockSpec((NG, 1), lambda i: (0, 0)),
        out_shape=jax.ShapeDtypeStruct((NG, 1), f32),
        scratch_shapes=[pltpu.VMEM((NG, 128), f32), pltpu.VMEM((NG, 8), f32)],
    )
    out1 = tc_d(S3, U3, dinv8, b1p, W3, bc3.reshape(1, 256),
                Wlin, blin.reshape(1, 1))
    return out1


# speedup vs baseline: 43.0385x; 1.2790x over previous
import functools

import jax
import jax.numpy as jnp
from jax import lax
from jax.experimental import pallas as pl
from jax.experimental.pallas import tpu as pltpu
from jax.experimental.pallas import tpu_sc as plsc

N = 10000
E = 640000
NG = 64
NPAD = 10240
NCORES = 2
NSUB = 16
TILES = NCORES * NSUB
CHUNK = 128
IDXBLK = 16
NBLK = 10
NCHUNK = IDXBLK * NBLK
EPAD = TILES * NCHUNK * CHUNK
RPT = NPAD // NSUB
RB = 1024
GRID = NPAD // RB


def _sc_mesh():
    return plsc.VectorSubcoreMesh(
        core_axis_name="c", subcore_axis_name="s",
        num_cores=NCORES, num_subcores=NSUB)


def _make_edge_pass(D, nbuf):
    ngrp = IDXBLK // nbuf

    @functools.partial(
        pl.kernel,
        out_type=jax.ShapeDtypeStruct((NCORES, NSUB, RPT, D), jnp.float32),
        mesh=_sc_mesh(),
        compiler_params=pltpu.CompilerParams(use_tc_tiling_on_sc=False),
        scratch_types=[
            pltpu.VMEM_SHARED((NPAD, D), jnp.float32),
            pltpu.VMEM((IDXBLK, CHUNK), jnp.int32),
            pltpu.VMEM((IDXBLK, CHUNK), jnp.int32),
        ] + [pltpu.VMEM((CHUNK, D), jnp.float32) for _ in range(nbuf)]
          + [pltpu.SemaphoreType.DMA for _ in range(nbuf)],
    )
    def edge_pass(table, srcp, dstp, zeros, out, accum, src_v, dst_v, *rest):
        bufs = rest[:nbuf]
        sems = rest[nbuf:]
        c = lax.axis_index("c")
        s = lax.axis_index("s")
        wid = c * NSUB + s
        pltpu.sync_copy(zeros, accum.at[pl.ds(s * RPT, RPT)])
        plsc.subcore_barrier()

        def blk_body(blk, _):
            pltpu.sync_copy(srcp.at[wid, blk], src_v)
            pltpu.sync_copy(dstp.at[wid, blk], dst_v)
            for b in range(nbuf):
                pltpu.async_copy(table.at[src_v.at[b]], bufs[b], sems[b])

            def grp(g, _):
                for b in range(nbuf):
                    j = g * nbuf + b
                    pltpu.make_async_copy(
                        table.at[src_v.at[j]], bufs[b], sems[b]).wait()
                    pltpu.sync_copy(bufs[b], accum.at[dst_v.at[j]], add=True)
                    pltpu.async_copy(
                        table.at[src_v.at[j + nbuf]], bufs[b], sems[b])
                return 0

            lax.fori_loop(0, ngrp - 1, grp, 0)
            for b in range(nbuf):
                j = (ngrp - 1) * nbuf + b
                pltpu.make_async_copy(
                    table.at[src_v.at[j]], bufs[b], sems[b]).wait()
                pltpu.sync_copy(bufs[b], accum.at[dst_v.at[j]], add=True)
            return 0

        lax.fori_loop(0, NBLK, blk_body, 0)
        plsc.subcore_barrier()
        pltpu.sync_copy(accum.at[pl.ds(s * RPT, RPT)], out.at[c, s])

    return edge_pass


@functools.partial(
    pl.kernel,
    out_type=jax.ShapeDtypeStruct((NCORES, NSUB, RPT, 8), jnp.float32),
    mesh=_sc_mesh(),
    compiler_params=pltpu.CompilerParams(use_tc_tiling_on_sc=False),
    scratch_types=[
        pltpu.VMEM_SHARED((NPAD, 8), jnp.float32),
        pltpu.VMEM((NCHUNK, CHUNK), jnp.int32),
        pltpu.VMEM((CHUNK, 8), jnp.float32),
    ],
)
def _deg_pass(dstp, zeros, ones, out, accum, dst_v, ones_v):
    c = lax.axis_index("c")
    s = lax.axis_index("s")
    wid = c * NSUB + s
    pltpu.sync_copy(zeros, accum.at[pl.ds(s * RPT, RPT)])
    pltpu.sync_copy(dstp.at[wid], dst_v)
    pltpu.sync_copy(ones, ones_v)
    plsc.subcore_barrier()

    def body(j, _):
        pltpu.sync_copy(ones_v, accum.at[dst_v.at[j]], add=True)
        return 0

    lax.fori_loop(0, NCHUNK, body, 0)
    plsc.subcore_barrier()
    pltpu.sync_copy(accum.at[pl.ds(s * RPT, RPT)], out.at[c, s])


def _tc_a_body(x_ref, w1_ref, degp_ref, u1_ref, dinv_ref):
    deg = degp_ref[0] + degp_ref[1] + 1.0
    dinv = lax.rsqrt(deg)
    dinv_ref[...] = dinv
    xw = jnp.dot(x_ref[...], w1_ref[...], preferred_element_type=jnp.float32)
    u1_ref[...] = xw * dinv[:, :1]


def _tc_b_body(sp_ref, u1_ref, dinv_ref, bc1_ref, u2_ref):
    dinv = dinv_ref[:, :1]
    t = dinv * (sp_ref[0] + sp_ref[1] + u1_ref[...]) + bc1_ref[...]
    u2_ref[...] = dinv * jnp.maximum(t, 0.0)


def _tc_c_body(sp_ref, u2_ref, dinv_ref, w2_ref, bc2_ref, u3_ref):
    dinv = dinv_ref[:, :1]
    t = dinv * (sp_ref[0] + sp_ref[1] + u2_ref[...])
    h = jnp.maximum(
        jnp.dot(t, w2_ref[...], preferred_element_type=jnp.float32)
        + bc2_ref[...], 0.0)
    u3_ref[...] = dinv * h


def _tc_d_body(sp_ref, u3_ref, dinv_ref, b1_ref, w3_ref, bc3_ref,
               wlin_ref, blin_ref, out_ref, g_acc, cnt_acc):
    i = pl.program_id(0)

    @pl.when(i == 0)
    def _():
        g_acc[...] = jnp.zeros_like(g_acc)
        cnt_acc[...] = jnp.zeros_like(cnt_acc)

    z = dinv_ref[:, :1] * (sp_ref[0] + sp_ref[1] + u3_ref[...])
    gids = lax.broadcasted_iota(jnp.int32, (1, NG), 1)
    onehot = (b1_ref[...] == gids).astype(jnp.float32)
    g_acc[...] += lax.dot_general(
        onehot, z, (((0,), (0,)), ((), ())),
        preferred_element_type=jnp.float32)
    cnt_acc[...] += lax.dot_general(
        onehot, jnp.ones((RB, 8), jnp.float32), (((0,), (0,)), ((), ())),
        preferred_element_type=jnp.float32)

    @pl.when(i == GRID - 1)
    def _():
        cnt = cnt_acc[:, :1]
        psum = jnp.dot(g_acc[...], w3_ref[...],
                       preferred_element_type=jnp.float32) + cnt * bc3_ref[...]
        pooled = psum / jnp.maximum(cnt, 1.0)
        out_ref[...] = jnp.dot(pooled, wlin_ref[...],
                               preferred_element_type=jnp.float32) + blin_ref[...]


def _row_spec(d):
    return pl.BlockSpec((RB, d), lambda i: (i, 0))


def _full_spec(shape):
    nd = len(shape)
    return pl.BlockSpec(shape, lambda i, _nd=nd: (0,) * _nd)


def _partial_spec(d):
    return pl.BlockSpec((NCORES, RB, d), lambda i: (0, i, 0))


def kernel(x1, edge_index1, b1, status, W1, bc1, W2, bc2, W3, bc3, Wlin, blin):
    f32 = jnp.float32
    src = edge_index1[0]
    dst = edge_index1[1]
    npad_extra = NPAD - N
    pad_e = EPAD - E
    padidx = (N + jnp.arange(pad_e, dtype=jnp.int32) % npad_extra)
    srcp = jnp.concatenate([src, padidx]).reshape(TILES, NBLK, IDXBLK, CHUNK)
    dstp = jnp.concatenate([dst, padidx]).reshape(TILES, NBLK, IDXBLK, CHUNK)
    dstp_flat = dstp.reshape(TILES, NCHUNK, CHUNK)
    xpad = jnp.concatenate([x1, jnp.zeros((npad_extra, x1.shape[1]), f32)])
    b1p = jnp.concatenate([b1, jnp.full((npad_extra,), NG, jnp.int32)])
    b1p = b1p.reshape(NPAD, 1)
    zeros8 = jnp.zeros((RPT, 8), f32)
    zeros64 = jnp.zeros((RPT, 64), f32)
    zeros128 = jnp.zeros((RPT, 128), f32)
    ones8 = jnp.ones((CHUNK, 8), f32)

    degp = _deg_pass(dstp_flat, zeros8, ones8).reshape(NCORES, NPAD, 8)

    tc_a = pl.pallas_call(
        _tc_a_body,
        grid=(GRID,),
        in_specs=[_row_spec(128), _full_spec((128, 64)), _partial_spec(8)],
        out_specs=[_row_spec(64), _row_spec(8)],
        out_shape=[jax.ShapeDtypeStruct((NPAD, 64), f32),
                   jax.ShapeDtypeStruct((NPAD, 8), f32)],
    )
    U1, dinv8 = tc_a(xpad, W1, degp)

    edge64 = _make_edge_pass(64, 4)
    edge128 = _make_edge_pass(128, 2)

    S1 = edge64(U1, srcp, dstp, zeros64).reshape(NCORES, NPAD, 64)

    tc_b = pl.pallas_call(
        _tc_b_body,
        grid=(GRID,),
        in_specs=[_partial_spec(64), _row_spec(64), _row_spec(8),
                  _full_spec((1, 64))],
        out_specs=_row_spec(64),
        out_shape=jax.ShapeDtypeStruct((NPAD, 64), f32),
    )
    U2 = tc_b(S1, U1, dinv8, bc1.reshape(1, 64))

    S2 = edge64(U2, srcp, dstp, zeros64).reshape(NCORES, NPAD, 64)

    tc_c = pl.pallas_call(
        _tc_c_body,
        grid=(GRID,),
        in_specs=[_partial_spec(64), _row_spec(64), _row_spec(8),
                  _full_spec((64, 128)), _full_spec((1, 128))],
        out_specs=_row_spec(128),
        out_shape=jax.ShapeDtypeStruct((NPAD, 128), f32),
    )
    U3 = tc_c(S2, U2, dinv8, W2, bc2.reshape(1, 128))

    S3 = edge128(U3, srcp, dstp, zeros128).reshape(NCORES, NPAD, 128)

    tc_d = pl.pallas_call(
        _tc_d_body,
        grid=(GRID,),
        in_specs=[_partial_spec(128), _row_spec(128), _row_spec(8),
                  pl.BlockSpec((RB, 1), lambda i: (i, 0)),
                  _full_spec((128, 256)), _full_spec((1, 256)),
                  _full_spec((256, 1)), _full_spec((1, 1))],
        out_specs=pl.BlockSpec((NG, 1), lambda i: (0, 0)),
        out_shape=jax.ShapeDtypeStruct((NG, 1), f32),
        scratch_shapes=[pltpu.VMEM((NG, 128), f32), pltpu.VMEM((NG, 8), f32)],
    )
    out1 = tc_d(S3, U3, dinv8, b1p, W3, bc3.reshape(1, 256),
                Wlin, blin.reshape(1, 1))
    return out1
